# baseline (speedup 1.0000x reference)
import jax
import jax.numpy as jnp
from jax.experimental import pallas as pl
from jax.experimental.pallas import tpu as pltpu


def _add_kernel(x_ref, e_ref, o_ref):
    o_ref[...] = x_ref[...] + e_ref[...][None, :, :]


def _add_rows(inputs, pos, row0, nrows, sblk):
    B, S, D = inputs.shape
    return pl.pallas_call(
        _add_kernel,
        grid=(nrows // sblk,),
        in_specs=[
            pl.BlockSpec((B, sblk, D), lambda i: (0, row0 // sblk + i, 0)),
            pl.BlockSpec((sblk, D), lambda i: (row0 // sblk + i, 0)),
        ],
        out_specs=pl.BlockSpec((B, sblk, D), lambda i: (0, i, 0)),
        out_shape=jax.ShapeDtypeStruct((B, nrows, D), inputs.dtype),
        compiler_params=pltpu.CompilerParams(
            dimension_semantics=("arbitrary",),
        ),
    )(inputs, pos)


def kernel(inputs, embeddings):
    B, S, D = inputs.shape
    pos = embeddings[:S]
    half = S // 2
    a = _add_rows(inputs, pos, 0, half, 512)
    b = _add_rows(inputs, pos, half, half, 512)
    return jnp.concatenate([a, b], axis=1)
